# Initial kernel scaffold; baseline (speedup 1.0000x reference)
import jax
import jax.numpy as jnp
from jax.experimental import pallas as pl


def kernel(x_in, table):
    raise NotImplementedError("write your pallas kernel here")


# speedup vs baseline: 4.1268x; 4.1268x over previous
import functools

import numpy as np
import jax
import jax.numpy as jnp
from jax import lax
from jax.experimental import pallas as pl
from jax.experimental.pallas import tpu as pltpu
from jax.experimental.pallas import tpu_sc as plsc

_EMBED_DIM = 16
_SEQ = 200
_NC = 2
_NS = 16
_NW = _NC * _NS
_R = 800


def _pos_encoding_np(max_pos: int, embed_dim: int) -> np.ndarray:
    pos = np.arange(max_pos)[:, np.newaxis]
    i = np.arange(embed_dim)[np.newaxis, :]
    angle_rates = 1.0 / np.power(10000, 2 * (i // 2) / np.float32(embed_dim))
    angle_rads = pos * angle_rates
    angle_rads[:, 0::2] = np.sin(angle_rads[:, 0::2])
    angle_rads[:, 1::2] = np.cos(angle_rads[:, 1::2])
    return angle_rads.astype(np.float32)


@functools.partial(jax.jit, static_argnames=("n_rows",))
def _sc_embed(x_flat, table, pos_tiled, *, n_rows: int):
    per_w = n_rows // _NW
    n_chunks = per_w // _R
    mesh = plsc.VectorSubcoreMesh(
        core_axis_name="c", subcore_axis_name="s",
        num_cores=_NC, num_subcores=_NS)

    @functools.partial(
        pl.kernel,
        out_type=jax.ShapeDtypeStruct((n_rows, _EMBED_DIM), jnp.float32),
        mesh=mesh,
        scratch_types=[
            pltpu.VMEM((_R,), jnp.int32),
            pltpu.VMEM((_R, _EMBED_DIM), jnp.float32),
            pltpu.VMEM((_R, _EMBED_DIM), jnp.float32),
            pltpu.SemaphoreType.DMA,
        ],
        compiler_params=pltpu.CompilerParams(use_tc_tiling_on_sc=False),
    )
    def k(idx_hbm, table_hbm, pos_hbm, out_hbm, idx_v, buf_v, pos_v, sem):
        wid = lax.axis_index("s") * _NC + lax.axis_index("c")
        base = wid * per_w
        pltpu.sync_copy(pos_hbm, pos_v)

        def chunk_body(c, carry):
            off = base + c * _R
            pltpu.sync_copy(idx_hbm.at[pl.ds(off, _R)], idx_v)
            pltpu.async_copy(table_hbm.at[idx_v], buf_v, sem).wait()

            def add_row(i, carry2):
                buf_v[i, :] = buf_v[i, :] + pos_v[i, :]
                return carry2

            lax.fori_loop(0, _R, add_row, 0, unroll=4)
            pltpu.sync_copy(buf_v, out_hbm.at[pl.ds(off, _R)])
            return carry

        lax.fori_loop(0, n_chunks, chunk_body, 0)

    return k(x_flat, table, pos_tiled)


def kernel(x_in, table):
    b, s = x_in.shape
    n_rows = b * s
    pos_tiled = jnp.asarray(
        np.tile(_pos_encoding_np(s, _EMBED_DIM), (_R // s, 1)))
    x_flat = x_in.reshape(n_rows).astype(jnp.int32)
    out = _sc_embed(x_flat, table, pos_tiled, n_rows=n_rows)
    return out.reshape(b, s, _EMBED_DIM)


# speedup vs baseline: 4.2793x; 1.0370x over previous
import functools

import numpy as np
import jax
import jax.numpy as jnp
from jax import lax
from jax.experimental import pallas as pl
from jax.experimental.pallas import tpu as pltpu
from jax.experimental.pallas import tpu_sc as plsc

_EMBED_DIM = 16
_SEQ = 200
_NC = 2
_NS = 16
_NW = _NC * _NS
_R = 800


def _pos_encoding_np(max_pos: int, embed_dim: int) -> np.ndarray:
    pos = np.arange(max_pos)[:, np.newaxis]
    i = np.arange(embed_dim)[np.newaxis, :]
    angle_rates = 1.0 / np.power(10000, 2 * (i // 2) / np.float32(embed_dim))
    angle_rads = pos * angle_rates
    angle_rads[:, 0::2] = np.sin(angle_rads[:, 0::2])
    angle_rads[:, 1::2] = np.cos(angle_rads[:, 1::2])
    return angle_rads.astype(np.float32)


@functools.partial(jax.jit, static_argnames=("n_rows",))
def _sc_embed(x_flat, table, pos_tiled, *, n_rows: int):
    per_w = n_rows // _NW
    n_chunks = per_w // _R
    mesh = plsc.VectorSubcoreMesh(
        core_axis_name="c", subcore_axis_name="s",
        num_cores=_NC, num_subcores=_NS)

    @functools.partial(
        pl.kernel,
        out_type=jax.ShapeDtypeStruct((n_rows, _EMBED_DIM), jnp.float32),
        mesh=mesh,
        scratch_types=[
            pltpu.VMEM((_R,), jnp.int32),
            pltpu.VMEM((_R, _EMBED_DIM), jnp.float32),
            pltpu.VMEM((_R, _EMBED_DIM), jnp.float32),
            pltpu.SemaphoreType.DMA,
        ],
        compiler_params=pltpu.CompilerParams(use_tc_tiling_on_sc=False),
    )
    def k(idx_hbm, table_hbm, pos_hbm, out_hbm, idx_v, buf_v, pos_v, sem):
        wid = lax.axis_index("s") * _NC + lax.axis_index("c")
        base = wid * per_w
        pltpu.sync_copy(pos_hbm, pos_v)

        def chunk_body(c, carry):
            off = base + c * _R
            pltpu.sync_copy(idx_hbm.at[pl.ds(off, _R)], idx_v)

            def prefill_row(i, carry2):
                buf_v[i, :] = pos_v[i, :]
                return carry2

            lax.fori_loop(0, _R, prefill_row, 0, unroll=4)
            pltpu.async_copy(table_hbm.at[idx_v], buf_v, sem, add=True).wait()
            pltpu.sync_copy(buf_v, out_hbm.at[pl.ds(off, _R)])
            return carry

        lax.fori_loop(0, n_chunks, chunk_body, 0)

    return k(x_flat, table, pos_tiled)


def kernel(x_in, table):
    b, s = x_in.shape
    n_rows = b * s
    pos_tiled = jnp.asarray(
        np.tile(_pos_encoding_np(s, _EMBED_DIM), (_R // s, 1)))
    x_flat = x_in.reshape(n_rows).astype(jnp.int32)
    out = _sc_embed(x_flat, table, pos_tiled, n_rows=n_rows)
    return out.reshape(b, s, _EMBED_DIM)


# speedup vs baseline: 5.7968x; 1.3546x over previous
import functools

import numpy as np
import jax
import jax.numpy as jnp
from jax import lax
from jax.experimental import pallas as pl
from jax.experimental.pallas import tpu as pltpu
from jax.experimental.pallas import tpu_sc as plsc

_EMBED_DIM = 16
_NC = 2
_NS = 16
_NW = _NC * _NS
_R = 800
_NBUF = 3


def _pos_encoding_np(max_pos: int, embed_dim: int) -> np.ndarray:
    pos = np.arange(max_pos)[:, np.newaxis]
    i = np.arange(embed_dim)[np.newaxis, :]
    angle_rates = 1.0 / np.power(10000, 2 * (i // 2) / np.float32(embed_dim))
    angle_rads = pos * angle_rates
    angle_rads[:, 0::2] = np.sin(angle_rads[:, 0::2])
    angle_rads[:, 1::2] = np.cos(angle_rads[:, 1::2])
    return angle_rads.astype(np.float32)


@functools.partial(jax.jit, static_argnames=("n_rows",))
def _sc_embed(x_flat, table, pos_tiled, *, n_rows: int):
    per_w = n_rows // _NW
    n_chunks = per_w // _R
    mesh = plsc.VectorSubcoreMesh(
        core_axis_name="c", subcore_axis_name="s",
        num_cores=_NC, num_subcores=_NS)

    @functools.partial(
        pl.kernel,
        out_type=jax.ShapeDtypeStruct((n_rows, _EMBED_DIM), jnp.float32),
        mesh=mesh,
        scratch_types=[
            pltpu.VMEM((per_w,), jnp.int32),
            pltpu.VMEM((_R, _EMBED_DIM), jnp.float32),
        ] + [pltpu.VMEM((_R, _EMBED_DIM), jnp.float32)] * _NBUF
          + [pltpu.SemaphoreType.DMA] * (2 * _NBUF),
        compiler_params=pltpu.CompilerParams(use_tc_tiling_on_sc=False),
    )
    def k(idx_hbm, table_hbm, pos_hbm, out_hbm, idx_all, pos_v, *bufs_sems):
        bufs = bufs_sems[:_NBUF]
        gsems = bufs_sems[_NBUF:2 * _NBUF]
        osems = bufs_sems[2 * _NBUF:]
        wid = lax.axis_index("s") * _NC + lax.axis_index("c")
        base = wid * per_w
        pltpu.sync_copy(idx_hbm.at[pl.ds(base, per_w)], idx_all)
        pltpu.sync_copy(pos_hbm, pos_v)

        def prefill(buf):
            def row(i, c2):
                buf[i, :] = pos_v[i, :]
                return c2
            lax.fori_loop(0, _R, row, 0, unroll=8)

        def fire_gather(c, b):
            pltpu.async_copy(
                table_hbm.at[idx_all.at[pl.ds(c * _R, _R)]],
                bufs[b], gsems[b], add=True)

        def fire_out(c, b):
            pltpu.async_copy(
                bufs[b], out_hbm.at[pl.ds(base + c * _R, _R)], osems[b])

        for b in range(_NBUF):
            prefill(bufs[b])
        for c in range(min(_NBUF, n_chunks)):
            fire_gather(c, c % _NBUF)

        for c in range(n_chunks):
            b = c % _NBUF
            pltpu.make_async_copy(
                table_hbm.at[idx_all.at[pl.ds(c * _R, _R)]],
                bufs[b], gsems[b]).wait()
            fire_out(c, b)
            if c + _NBUF < n_chunks:
                pltpu.make_async_copy(
                    bufs[b], out_hbm.at[pl.ds(base + c * _R, _R)],
                    osems[b]).wait()
                prefill(bufs[b])
                fire_gather(c + _NBUF, b)

        for c in range(max(0, n_chunks - _NBUF), n_chunks):
            b = c % _NBUF
            pltpu.make_async_copy(
                bufs[b], out_hbm.at[pl.ds(base + c * _R, _R)],
                osems[b]).wait()

    return k(x_flat, table, pos_tiled)


def kernel(x_in, table):
    b, s = x_in.shape
    n_rows = b * s
    pos_tiled = jnp.asarray(
        np.tile(_pos_encoding_np(s, _EMBED_DIM), (_R // s, 1)))
    x_flat = x_in.reshape(n_rows).astype(jnp.int32)
    out = _sc_embed(x_flat, table, pos_tiled, n_rows=n_rows)
    return out.reshape(b, s, _EMBED_DIM)
